# baseline (speedup 1.0000x reference)
import functools

import jax
import jax.numpy as jnp
from jax import lax
from jax.experimental import pallas as pl
from jax.experimental.pallas import tpu as pltpu
from jax.experimental.pallas import tpu_sc as plsc

_N = 10000
_E = 320000
_D = 128
_L = 16
_NC = 2
_NS = 16
_NW = _NC * _NS
_RB = 2560
_BA = 72
_EA = _BA * _RB
_EB = _E - _EA
_BB = _EB // _RB
_C = 80
_EPA = _EA // _NW
_EPB = _EB // _NW
_GS = 8
_GMAX = 9
_NA = 10240
_RPS = _NA // _NS

_DN = (((1,), (1,)), ((), ()))


def _proj_a_body(nodes_ref, ev_ref, wn_ref, bn_ref, we_ref, be_ref,
                 vnode_ref, vedge_ref):
    @pl.when(pl.program_id(0) == 0)
    def _():
        vnode_ref[...] = (
            lax.dot_general(nodes_ref[...], wn_ref[...], _DN,
                            preferred_element_type=jnp.float32) + bn_ref[...]
        )
    vedge_ref[...] = (
        lax.dot_general(ev_ref[...], we_ref[...], _DN,
                        preferred_element_type=jnp.float32) + be_ref[...]
    )


def _proj_a(nodes, edges_values, wn, bn, we, be):
    return pl.pallas_call(
        _proj_a_body,
        grid=(_BA,),
        in_specs=[
            pl.BlockSpec((_N, _D), lambda i: (0, 0)),
            pl.BlockSpec((_RB, _D), lambda i: (i, 0)),
            pl.BlockSpec((_D, _D), lambda i: (0, 0)),
            pl.BlockSpec((1, _D), lambda i: (0, 0)),
            pl.BlockSpec((_D, _D), lambda i: (0, 0)),
            pl.BlockSpec((1, _D), lambda i: (0, 0)),
        ],
        out_specs=[
            pl.BlockSpec((_N, _D), lambda i: (0, 0)),
            pl.BlockSpec((_RB, _D), lambda i: (i, 0)),
        ],
        out_shape=[
            jax.ShapeDtypeStruct((_N, _D), jnp.float32),
            jax.ShapeDtypeStruct((_EA, _D), jnp.float32),
        ],
    )(nodes, edges_values, wn, bn, we, be)


def _proj_b_body(ev_ref, we_ref, be_ref, vedge_ref):
    vedge_ref[...] = (
        lax.dot_general(ev_ref[...], we_ref[...], _DN,
                        preferred_element_type=jnp.float32) + be_ref[...]
    )


def _proj_b(edges_values, we, be):
    return pl.pallas_call(
        _proj_b_body,
        grid=(_BB,),
        in_specs=[
            pl.BlockSpec((_RB, _D), lambda i: (i + _BA, 0)),
            pl.BlockSpec((_D, _D), lambda i: (0, 0)),
            pl.BlockSpec((1, _D), lambda i: (0, 0)),
        ],
        out_specs=pl.BlockSpec((_RB, _D), lambda i: (i, 0)),
        out_shape=jax.ShapeDtypeStruct((_EB, _D), jnp.float32),
    )(edges_values, we, be)


_SC_SCRATCH = [
    pltpu.VMEM((2, _GS, _C), jnp.int32),
    pltpu.VMEM((2, _C, _D), jnp.float32),
    pltpu.VMEM((_C, _D), jnp.float32),
    pltpu.VMEM((_C, _D), jnp.float32),
    pltpu.VMEM_SHARED((_NA, _D), jnp.float32),
    pltpu.SemaphoreType.DMA((2,)),
    pltpu.SemaphoreType.DMA((2,)),
    pltpu.SemaphoreType.DMA((2,)),
    pltpu.SemaphoreType.DMA((2,)),
    pltpu.SemaphoreType.DMA((2,)),
]


def _sc_pipeline(half, nchunk, ngroups, epw, half_base,
                 vnode_hbm, dst_hbm, vedge_hbm, oedge_hbm,
                 idx, gbuf, ebufs, acc, sem_g, sem_e, sem_o, sem_s,
                 sem_i, wid):
    ebase = wid * epw

    def _start_idx(s, g):
        pltpu.async_copy(dst_hbm.at[half, wid, g], idx.at[s], sem_i.at[s])

    def _wait_idx(s, g):
        pltpu.make_async_copy(dst_hbm.at[half, wid, g], idx.at[s],
                              sem_i.at[s]).wait()

    def _start_in(s, i, gs, j):
        pltpu.async_copy(vnode_hbm.at[idx.at[gs, j]], gbuf.at[s],
                         sem_g.at[s])
        pltpu.async_copy(vedge_hbm.at[pl.ds(ebase + i * _C, _C)],
                         ebufs[s], sem_e.at[s])

    def _wait_in(s, i, gs, j):
        pltpu.make_async_copy(vnode_hbm.at[idx.at[gs, j]], gbuf.at[s],
                              sem_g.at[s]).wait()
        pltpu.make_async_copy(vedge_hbm.at[pl.ds(ebase + i * _C, _C)],
                              ebufs[s], sem_e.at[s]).wait()

    def _mul(s):
        ebuf = ebufs[s]

        @plsc.parallel_loop(0, _C, step=1, unroll=4)
        def _mrow(r2):
            for j in range(_D // _L):
                sl = pl.ds(j * _L, _L)
                gbuf[s, r2, sl] = gbuf[s, r2, sl] * ebuf[r2, sl]

    def _start_out(s, i, gs, j):
        pltpu.async_copy(gbuf.at[s],
                         oedge_hbm.at[pl.ds(half_base + ebase + i * _C, _C)],
                         sem_o.at[s])
        pltpu.async_copy(gbuf.at[s], acc.at[idx.at[gs, j]], sem_s.at[s],
                         add=True)

    def _wait_out(s, i, gs, j):
        pltpu.make_async_copy(gbuf.at[s],
                              oedge_hbm.at[pl.ds(half_base + ebase + i * _C,
                                                 _C)],
                              sem_o.at[s]).wait()
        pltpu.make_async_copy(gbuf.at[s], acc.at[idx.at[gs, j]],
                              sem_s.at[s]).wait()

    pltpu.sync_copy(dst_hbm.at[half, wid, 0], idx.at[0])
    _start_in(0, 0, 0, 0)

    def _chunk_step(i, s):
        ns = 1 - s
        g = lax.div(i, _GS)
        j = lax.rem(i, _GS)
        gs = lax.rem(g, 2)
        ngs = 1 - gs

        @pl.when(i >= 1)
        def _():
            _wait_out(ns, i - 1, lax.rem(lax.div(i - 1, _GS), 2),
                      lax.rem(i - 1, _GS))

        @pl.when(jnp.logical_and(j == 1, g < ngroups - 1))
        def _():
            _start_idx(ngs, g + 1)

        @pl.when(j == _GS - 1)
        def _():
            _wait_idx(ngs, g + 1)

        nxt_gs = lax.rem(lax.div(i + 1, _GS), 2)
        _start_in(ns, i + 1, nxt_gs, lax.rem(i + 1, _GS))
        _wait_in(s, i, gs, j)
        _mul(s)
        _start_out(s, i, gs, j)

    nbody = nchunk - 1

    def _pair(p, carry):
        _chunk_step(2 * p, 0)
        _chunk_step(2 * p + 1, 1)
        return carry

    lax.fori_loop(0, nbody // 2, _pair, 0)
    if nbody % 2:
        _chunk_step(nbody - 1, (nbody - 1) % 2)
    last = nchunk - 1
    s_last = last % 2
    g_last = (last // _GS) % 2
    j_last = last % _GS
    _wait_in(s_last, last, g_last, j_last)
    _mul(s_last)
    _start_out(s_last, last, g_last, j_last)
    _wait_out(1 - s_last, last - 1, ((last - 1) // _GS) % 2, (last - 1) % _GS)
    _wait_out(s_last, last, g_last, j_last)


@functools.partial(
    pl.kernel,
    mesh=plsc.VectorSubcoreMesh(core_axis_name="c", subcore_axis_name="s"),
    out_type=[
        jax.ShapeDtypeStruct((_E, _D), jnp.float32),
        jax.ShapeDtypeStruct((_NC, _NA, _D), jnp.float32),
    ],
    scratch_types=_SC_SCRATCH,
)
def _sc_first(vnode_hbm, dst_hbm, vedge_hbm, oedge_hbm, part_hbm,
              idx, gbuf, eb0, eb1, acc, sem_g, sem_e, sem_o, sem_s, sem_i):
    cid = lax.axis_index("c")
    sid = lax.axis_index("s")
    wid = cid * _NS + sid
    row0 = sid * _RPS

    for sl in range(2):
        def _zrow(i, carry, _sl=sl):
            for j in range(_D // _L):
                gbuf[_sl, i, pl.ds(j * _L, _L)] = jnp.zeros((_L,), jnp.float32)
            return carry

        lax.fori_loop(0, _C, _zrow, 0)
    for r in range(_RPS // _C):
        pltpu.sync_copy(gbuf.at[r % 2], acc.at[pl.ds(row0 + r * _C, _C)])
    plsc.subcore_barrier()

    _sc_pipeline(0, _EPA // _C, _EPA // (_GS * _C), _EPA, 0,
                 vnode_hbm, dst_hbm, vedge_hbm, oedge_hbm,
                 idx, gbuf, (eb0, eb1), acc, sem_g, sem_e, sem_o, sem_s,
                 sem_i, wid)

    plsc.subcore_barrier()
    pltpu.sync_copy(acc.at[pl.ds(row0, _RPS)],
                    part_hbm.at[cid, pl.ds(row0, _RPS)])


@functools.partial(
    pl.kernel,
    mesh=plsc.VectorSubcoreMesh(core_axis_name="c", subcore_axis_name="s"),
    out_type=jax.ShapeDtypeStruct((_NC, _NA, _D), jnp.float32),
    scratch_types=_SC_SCRATCH,
)
def _sc_second(vnode_hbm, dst_hbm, vedge_hbm, part_hbm, oedge_hbm,
               partb_hbm,
               idx, gbuf, eb0, eb1, acc, sem_g, sem_e, sem_o, sem_s, sem_i):
    cid = lax.axis_index("c")
    sid = lax.axis_index("s")
    wid = cid * _NS + sid
    row0 = sid * _RPS

    pltpu.sync_copy(part_hbm.at[cid, pl.ds(row0, _RPS)],
                    acc.at[pl.ds(row0, _RPS)])
    plsc.subcore_barrier()

    _sc_pipeline(1, _EPB // _C, (_EPB // _C + _GS - 1) // _GS, _EPB, _EA,
                 vnode_hbm, dst_hbm, vedge_hbm, oedge_hbm,
                 idx, gbuf, (eb0, eb1), acc, sem_g, sem_e, sem_o, sem_s,
                 sem_i, wid)

    plsc.subcore_barrier()
    pltpu.sync_copy(acc.at[pl.ds(row0, _RPS)],
                    partb_hbm.at[cid, pl.ds(row0, _RPS)])


def _add_body(p_ref, o_ref):
    o_ref[...] = p_ref[0, :_N, :] + p_ref[1, :_N, :]


def _final_add(partials):
    return pl.pallas_call(
        _add_body,
        out_shape=jax.ShapeDtypeStruct((_N, _D), jnp.float32),
    )(partials)


def kernel(nodes, edges_index, edges_values,
           WQ_node_w, WQ_node_b, WQ_edge_w, WQ_edge_b,
           WK_node_w, WK_node_b, WK_edge_w, WK_edge_b,
           WV_node_w, WV_node_b, WV_edge_w, WV_edge_b):
    dst = edges_index[1].astype(jnp.int32)
    pad = _GMAX * _GS * _C
    dst_a = jnp.pad(dst[:_EA].reshape(_NW, _EPA), ((0, 0), (0, pad - _EPA)))
    dst_b = jnp.pad(dst[_EA:].reshape(_NW, _EPB), ((0, 0), (0, pad - _EPB)))
    dst_all = jnp.stack([dst_a, dst_b]).reshape(2, _NW, _GMAX, _GS, _C)
    bn = WV_node_b.reshape(1, _D)
    be = WV_edge_b.reshape(1, _D)
    v_node, ve_a = _proj_a(nodes, edges_values, WV_node_w, bn, WV_edge_w, be)
    ve_b = _proj_b(edges_values, WV_edge_w, be)
    oedge_half, part_a = _sc_first(v_node, dst_all, ve_a)
    oedge_ref = jax.new_ref(oedge_half)
    part_b = _sc_second(v_node, dst_all, ve_b, part_a, oedge_ref)
    output_edges = jax.freeze(oedge_ref)
    output_nodes = _final_add(part_b)
    return (output_nodes, output_edges)


# speedup vs baseline: 1.0119x; 1.0119x over previous
import functools

import jax
import jax.numpy as jnp
from jax import lax
from jax.experimental import pallas as pl
from jax.experimental.pallas import tpu as pltpu
from jax.experimental.pallas import tpu_sc as plsc

_N = 10000
_E = 320000
_D = 128
_L = 16
_NC = 2
_NS = 16
_NW = _NC * _NS
_RB = 2560
_BA = 64
_EA = _BA * _RB
_EB = _E - _EA
_BB = _EB // _RB
_C = 80
_EPA = _EA // _NW
_EPB = _EB // _NW
_GS = 8
_GMAX = 9
_NA = 10240
_RPS = _NA // _NS

_DN = (((1,), (1,)), ((), ()))


def _proj_a_body(nodes_ref, ev_ref, wn_ref, bn_ref, we_ref, be_ref,
                 vnode_ref, vedge_ref):
    @pl.when(pl.program_id(0) == 0)
    def _():
        vnode_ref[...] = (
            lax.dot_general(nodes_ref[...], wn_ref[...], _DN,
                            preferred_element_type=jnp.float32) + bn_ref[...]
        )
    vedge_ref[...] = (
        lax.dot_general(ev_ref[...], we_ref[...], _DN,
                        preferred_element_type=jnp.float32) + be_ref[...]
    )


def _proj_a(nodes, edges_values, wn, bn, we, be):
    return pl.pallas_call(
        _proj_a_body,
        grid=(_BA,),
        in_specs=[
            pl.BlockSpec((_N, _D), lambda i: (0, 0)),
            pl.BlockSpec((_RB, _D), lambda i: (i, 0)),
            pl.BlockSpec((_D, _D), lambda i: (0, 0)),
            pl.BlockSpec((1, _D), lambda i: (0, 0)),
            pl.BlockSpec((_D, _D), lambda i: (0, 0)),
            pl.BlockSpec((1, _D), lambda i: (0, 0)),
        ],
        out_specs=[
            pl.BlockSpec((_N, _D), lambda i: (0, 0)),
            pl.BlockSpec((_RB, _D), lambda i: (i, 0)),
        ],
        out_shape=[
            jax.ShapeDtypeStruct((_N, _D), jnp.float32),
            jax.ShapeDtypeStruct((_EA, _D), jnp.float32),
        ],
    )(nodes, edges_values, wn, bn, we, be)


def _proj_b_body(ev_ref, we_ref, be_ref, vedge_ref):
    vedge_ref[...] = (
        lax.dot_general(ev_ref[...], we_ref[...], _DN,
                        preferred_element_type=jnp.float32) + be_ref[...]
    )


def _proj_b(edges_values, we, be):
    return pl.pallas_call(
        _proj_b_body,
        grid=(_BB,),
        in_specs=[
            pl.BlockSpec((_RB, _D), lambda i: (i + _BA, 0)),
            pl.BlockSpec((_D, _D), lambda i: (0, 0)),
            pl.BlockSpec((1, _D), lambda i: (0, 0)),
        ],
        out_specs=pl.BlockSpec((_RB, _D), lambda i: (i, 0)),
        out_shape=jax.ShapeDtypeStruct((_EB, _D), jnp.float32),
    )(edges_values, we, be)


_SC_SCRATCH = [
    pltpu.VMEM((2, _GS, _C), jnp.int32),
    pltpu.VMEM((2, _C, _D), jnp.float32),
    pltpu.VMEM((_C, _D), jnp.float32),
    pltpu.VMEM((_C, _D), jnp.float32),
    pltpu.VMEM_SHARED((_NA, _D), jnp.float32),
    pltpu.SemaphoreType.DMA((2,)),
    pltpu.SemaphoreType.DMA((2,)),
    pltpu.SemaphoreType.DMA((2,)),
    pltpu.SemaphoreType.DMA((2,)),
    pltpu.SemaphoreType.DMA((2,)),
]


def _sc_pipeline(half, nchunk, ngroups, epw, half_base,
                 vnode_hbm, dst_hbm, vedge_hbm, oedge_hbm,
                 idx, gbuf, ebufs, acc, sem_g, sem_e, sem_o, sem_s,
                 sem_i, wid):
    ebase = wid * epw

    def _start_idx(s, g):
        pltpu.async_copy(dst_hbm.at[half, wid, g], idx.at[s], sem_i.at[s])

    def _wait_idx(s, g):
        pltpu.make_async_copy(dst_hbm.at[half, wid, g], idx.at[s],
                              sem_i.at[s]).wait()

    def _start_in(s, i, gs, j):
        pltpu.async_copy(vnode_hbm.at[idx.at[gs, j]], gbuf.at[s],
                         sem_g.at[s])
        pltpu.async_copy(vedge_hbm.at[pl.ds(ebase + i * _C, _C)],
                         ebufs[s], sem_e.at[s])

    def _wait_in(s, i, gs, j):
        pltpu.make_async_copy(vnode_hbm.at[idx.at[gs, j]], gbuf.at[s],
                              sem_g.at[s]).wait()
        pltpu.make_async_copy(vedge_hbm.at[pl.ds(ebase + i * _C, _C)],
                              ebufs[s], sem_e.at[s]).wait()

    def _mul(s):
        ebuf = ebufs[s]

        @plsc.parallel_loop(0, _C, step=1, unroll=4)
        def _mrow(r2):
            for j in range(_D // _L):
                sl = pl.ds(j * _L, _L)
                gbuf[s, r2, sl] = gbuf[s, r2, sl] * ebuf[r2, sl]

    def _start_out(s, i, gs, j):
        pltpu.async_copy(gbuf.at[s],
                         oedge_hbm.at[pl.ds(half_base + ebase + i * _C, _C)],
                         sem_o.at[s])
        pltpu.async_copy(gbuf.at[s], acc.at[idx.at[gs, j]], sem_s.at[s],
                         add=True)

    def _wait_out(s, i, gs, j):
        pltpu.make_async_copy(gbuf.at[s],
                              oedge_hbm.at[pl.ds(half_base + ebase + i * _C,
                                                 _C)],
                              sem_o.at[s]).wait()
        pltpu.make_async_copy(gbuf.at[s], acc.at[idx.at[gs, j]],
                              sem_s.at[s]).wait()

    pltpu.sync_copy(dst_hbm.at[half, wid, 0], idx.at[0])
    _start_in(0, 0, 0, 0)

    def _chunk_step(i, s):
        ns = 1 - s
        g = lax.div(i, _GS)
        j = lax.rem(i, _GS)
        gs = lax.rem(g, 2)
        ngs = 1 - gs

        @pl.when(i >= 1)
        def _():
            _wait_out(ns, i - 1, lax.rem(lax.div(i - 1, _GS), 2),
                      lax.rem(i - 1, _GS))

        @pl.when(jnp.logical_and(j == 1, g < ngroups - 1))
        def _():
            _start_idx(ngs, g + 1)

        @pl.when(j == _GS - 1)
        def _():
            _wait_idx(ngs, g + 1)

        nxt_gs = lax.rem(lax.div(i + 1, _GS), 2)
        _start_in(ns, i + 1, nxt_gs, lax.rem(i + 1, _GS))
        _wait_in(s, i, gs, j)
        _mul(s)
        _start_out(s, i, gs, j)

    nbody = nchunk - 1

    def _pair(p, carry):
        _chunk_step(2 * p, 0)
        _chunk_step(2 * p + 1, 1)
        return carry

    lax.fori_loop(0, nbody // 2, _pair, 0)
    if nbody % 2:
        _chunk_step(nbody - 1, (nbody - 1) % 2)
    last = nchunk - 1
    s_last = last % 2
    g_last = (last // _GS) % 2
    j_last = last % _GS
    _wait_in(s_last, last, g_last, j_last)
    _mul(s_last)
    _start_out(s_last, last, g_last, j_last)
    _wait_out(1 - s_last, last - 1, ((last - 1) // _GS) % 2, (last - 1) % _GS)
    _wait_out(s_last, last, g_last, j_last)


@functools.partial(
    pl.kernel,
    mesh=plsc.VectorSubcoreMesh(core_axis_name="c", subcore_axis_name="s"),
    out_type=[
        jax.ShapeDtypeStruct((_E, _D), jnp.float32),
        jax.ShapeDtypeStruct((_NC, _NA, _D), jnp.float32),
    ],
    scratch_types=_SC_SCRATCH,
)
def _sc_first(vnode_hbm, dst_hbm, vedge_hbm, oedge_hbm, part_hbm,
              idx, gbuf, eb0, eb1, acc, sem_g, sem_e, sem_o, sem_s, sem_i):
    cid = lax.axis_index("c")
    sid = lax.axis_index("s")
    wid = cid * _NS + sid
    row0 = sid * _RPS

    for sl in range(2):
        def _zrow(i, carry, _sl=sl):
            for j in range(_D // _L):
                gbuf[_sl, i, pl.ds(j * _L, _L)] = jnp.zeros((_L,), jnp.float32)
            return carry

        lax.fori_loop(0, _C, _zrow, 0)
    for r in range(_RPS // _C):
        pltpu.sync_copy(gbuf.at[r % 2], acc.at[pl.ds(row0 + r * _C, _C)])
    plsc.subcore_barrier()

    _sc_pipeline(0, _EPA // _C, _EPA // (_GS * _C), _EPA, 0,
                 vnode_hbm, dst_hbm, vedge_hbm, oedge_hbm,
                 idx, gbuf, (eb0, eb1), acc, sem_g, sem_e, sem_o, sem_s,
                 sem_i, wid)

    plsc.subcore_barrier()
    pltpu.sync_copy(acc.at[pl.ds(row0, _RPS)],
                    part_hbm.at[cid, pl.ds(row0, _RPS)])


@functools.partial(
    pl.kernel,
    mesh=plsc.VectorSubcoreMesh(core_axis_name="c", subcore_axis_name="s"),
    out_type=jax.ShapeDtypeStruct((_NC, _NA, _D), jnp.float32),
    scratch_types=_SC_SCRATCH,
)
def _sc_second(vnode_hbm, dst_hbm, vedge_hbm, part_hbm, oedge_hbm,
               partb_hbm,
               idx, gbuf, eb0, eb1, acc, sem_g, sem_e, sem_o, sem_s, sem_i):
    cid = lax.axis_index("c")
    sid = lax.axis_index("s")
    wid = cid * _NS + sid
    row0 = sid * _RPS

    pltpu.sync_copy(part_hbm.at[cid, pl.ds(row0, _RPS)],
                    acc.at[pl.ds(row0, _RPS)])
    plsc.subcore_barrier()

    _sc_pipeline(1, _EPB // _C, (_EPB // _C + _GS - 1) // _GS, _EPB, _EA,
                 vnode_hbm, dst_hbm, vedge_hbm, oedge_hbm,
                 idx, gbuf, (eb0, eb1), acc, sem_g, sem_e, sem_o, sem_s,
                 sem_i, wid)

    plsc.subcore_barrier()
    pltpu.sync_copy(acc.at[pl.ds(row0, _RPS)],
                    partb_hbm.at[cid, pl.ds(row0, _RPS)])


def _add_body(p_ref, o_ref):
    o_ref[...] = p_ref[0, :_N, :] + p_ref[1, :_N, :]


def _final_add(partials):
    return pl.pallas_call(
        _add_body,
        out_shape=jax.ShapeDtypeStruct((_N, _D), jnp.float32),
    )(partials)


def kernel(nodes, edges_index, edges_values,
           WQ_node_w, WQ_node_b, WQ_edge_w, WQ_edge_b,
           WK_node_w, WK_node_b, WK_edge_w, WK_edge_b,
           WV_node_w, WV_node_b, WV_edge_w, WV_edge_b):
    dst = edges_index[1].astype(jnp.int32)
    pad = _GMAX * _GS * _C
    dst_a = jnp.pad(dst[:_EA].reshape(_NW, _EPA), ((0, 0), (0, pad - _EPA)))
    dst_b = jnp.pad(dst[_EA:].reshape(_NW, _EPB), ((0, 0), (0, pad - _EPB)))
    dst_all = jnp.stack([dst_a, dst_b]).reshape(2, _NW, _GMAX, _GS, _C)
    bn = WV_node_b.reshape(1, _D)
    be = WV_edge_b.reshape(1, _D)
    v_node, ve_a = _proj_a(nodes, edges_values, WV_node_w, bn, WV_edge_w, be)
    ve_b = _proj_b(edges_values, WV_edge_w, be)
    oedge_half, part_a = _sc_first(v_node, dst_all, ve_a)
    oedge_ref = jax.new_ref(oedge_half)
    part_b = _sc_second(v_node, dst_all, ve_b, part_a, oedge_ref)
    output_edges = jax.freeze(oedge_ref)
    output_nodes = _final_add(part_b)
    return (output_nodes, output_edges)


# speedup vs baseline: 1.0505x; 1.0381x over previous
import functools

import jax
import jax.numpy as jnp
from jax import lax
from jax.experimental import pallas as pl
from jax.experimental.pallas import tpu as pltpu
from jax.experimental.pallas import tpu_sc as plsc

_N = 10000
_E = 320000
_D = 128
_L = 16
_NC = 2
_NS = 16
_NW = _NC * _NS
_RB = 2560
_BA = 64
_EA = _BA * _RB
_EB = _E - _EA
_BB = _EB // _RB
_C = 80
_EPA = _EA // _NW
_EPB = _EB // _NW
_GS = 8
_GMAX = 9
_NA = 10240
_RPS = _NA // _NS

_DN = (((1,), (1,)), ((), ()))


def _proj_a_body(nodes_ref, ev_ref, wn_ref, bn_ref, we_ref, be_ref,
                 vnode_ref, vedge_ref):
    @pl.when(pl.program_id(0) == 0)
    def _():
        vnode_ref[...] = (
            lax.dot_general(nodes_ref[...], wn_ref[...], _DN,
                            preferred_element_type=jnp.float32) + bn_ref[...]
        )
    vedge_ref[...] = (
        lax.dot_general(ev_ref[...], we_ref[...], _DN,
                        preferred_element_type=jnp.float32) + be_ref[...]
    )


def _proj_a(nodes, edges_values, wn, bn, we, be):
    return pl.pallas_call(
        _proj_a_body,
        grid=(_BA,),
        in_specs=[
            pl.BlockSpec((_N, _D), lambda i: (0, 0)),
            pl.BlockSpec((_RB, _D), lambda i: (i, 0)),
            pl.BlockSpec((_D, _D), lambda i: (0, 0)),
            pl.BlockSpec((1, _D), lambda i: (0, 0)),
            pl.BlockSpec((_D, _D), lambda i: (0, 0)),
            pl.BlockSpec((1, _D), lambda i: (0, 0)),
        ],
        out_specs=[
            pl.BlockSpec((_N, _D), lambda i: (0, 0)),
            pl.BlockSpec((_RB, _D), lambda i: (i, 0)),
        ],
        out_shape=[
            jax.ShapeDtypeStruct((_N, _D), jnp.float32),
            jax.ShapeDtypeStruct((_EA, _D), jnp.float32),
        ],
    )(nodes, edges_values, wn, bn, we, be)


def _proj_b_body(ev_ref, we_ref, be_ref, vedge_ref):
    vedge_ref[...] = (
        lax.dot_general(ev_ref[...], we_ref[...], _DN,
                        preferred_element_type=jnp.float32) + be_ref[...]
    )


def _proj_b(edges_values, we, be):
    return pl.pallas_call(
        _proj_b_body,
        grid=(_BB,),
        in_specs=[
            pl.BlockSpec((_RB, _D), lambda i: (i + _BA, 0)),
            pl.BlockSpec((_D, _D), lambda i: (0, 0)),
            pl.BlockSpec((1, _D), lambda i: (0, 0)),
        ],
        out_specs=pl.BlockSpec((_RB, _D), lambda i: (i, 0)),
        out_shape=jax.ShapeDtypeStruct((_EB, _D), jnp.float32),
    )(edges_values, we, be)


_SC_SCRATCH = [
    pltpu.VMEM((2, _GS, _C), jnp.int32),
    pltpu.VMEM((2, _C, _D), jnp.float32),
    pltpu.VMEM((_C, _D), jnp.float32),
    pltpu.VMEM((_C, _D), jnp.float32),
    pltpu.VMEM_SHARED((_NA, _D), jnp.float32),
    pltpu.SemaphoreType.DMA((2,)),
    pltpu.SemaphoreType.DMA((2,)),
    pltpu.SemaphoreType.DMA((2,)),
    pltpu.SemaphoreType.DMA((2,)),
    pltpu.SemaphoreType.DMA((2,)),
]


def _sc_pipeline(nchunk, ngroups, epw, half_base,
                 vnode_hbm, dst_hbm, vedge_hbm, oedge_hbm,
                 idx, gbuf, ebufs, acc, sem_g, sem_e, sem_o, sem_s,
                 sem_i, wid):
    ebase = wid * epw

    def _start_idx(s, g):
        pltpu.async_copy(dst_hbm.at[wid, g], idx.at[s], sem_i.at[s])

    def _wait_idx(s, g):
        pltpu.make_async_copy(dst_hbm.at[wid, g], idx.at[s],
                              sem_i.at[s]).wait()

    def _start_in(s, i, gs, j):
        pltpu.async_copy(vnode_hbm.at[idx.at[gs, j]], gbuf.at[s],
                         sem_g.at[s])
        pltpu.async_copy(vedge_hbm.at[pl.ds(ebase + i * _C, _C)],
                         ebufs[s], sem_e.at[s])

    def _wait_in(s, i, gs, j):
        pltpu.make_async_copy(vnode_hbm.at[idx.at[gs, j]], gbuf.at[s],
                              sem_g.at[s]).wait()
        pltpu.make_async_copy(vedge_hbm.at[pl.ds(ebase + i * _C, _C)],
                              ebufs[s], sem_e.at[s]).wait()

    def _mul(s):
        ebuf = ebufs[s]

        @plsc.parallel_loop(0, _C, step=1, unroll=4)
        def _mrow(r2):
            for j in range(_D // _L):
                sl = pl.ds(j * _L, _L)
                gbuf[s, r2, sl] = gbuf[s, r2, sl] * ebuf[r2, sl]

    def _start_out(s, i, gs, j):
        pltpu.async_copy(gbuf.at[s],
                         oedge_hbm.at[pl.ds(half_base + ebase + i * _C, _C)],
                         sem_o.at[s])
        pltpu.async_copy(gbuf.at[s], acc.at[idx.at[gs, j]], sem_s.at[s],
                         add=True)

    def _wait_out(s, i, gs, j):
        pltpu.make_async_copy(gbuf.at[s],
                              oedge_hbm.at[pl.ds(half_base + ebase + i * _C,
                                                 _C)],
                              sem_o.at[s]).wait()
        pltpu.make_async_copy(gbuf.at[s], acc.at[idx.at[gs, j]],
                              sem_s.at[s]).wait()

    pltpu.sync_copy(dst_hbm.at[wid, 0], idx.at[0])
    _start_in(0, 0, 0, 0)

    def _chunk_step(i, s):
        ns = 1 - s
        g = lax.div(i, _GS)
        j = lax.rem(i, _GS)
        gs = lax.rem(g, 2)
        ngs = 1 - gs

        @pl.when(i >= 1)
        def _():
            _wait_out(ns, i - 1, lax.rem(lax.div(i - 1, _GS), 2),
                      lax.rem(i - 1, _GS))

        @pl.when(jnp.logical_and(j == 1, g < ngroups - 1))
        def _():
            _start_idx(ngs, g + 1)

        @pl.when(j == _GS - 1)
        def _():
            _wait_idx(ngs, g + 1)

        nxt_gs = lax.rem(lax.div(i + 1, _GS), 2)
        _start_in(ns, i + 1, nxt_gs, lax.rem(i + 1, _GS))
        _wait_in(s, i, gs, j)
        _mul(s)
        _start_out(s, i, gs, j)

    nbody = nchunk - 1

    def _pair(p, carry):
        _chunk_step(2 * p, 0)
        _chunk_step(2 * p + 1, 1)
        return carry

    lax.fori_loop(0, nbody // 2, _pair, 0)
    if nbody % 2:
        _chunk_step(nbody - 1, (nbody - 1) % 2)
    last = nchunk - 1
    s_last = last % 2
    g_last = (last // _GS) % 2
    j_last = last % _GS
    _wait_in(s_last, last, g_last, j_last)
    _mul(s_last)
    _start_out(s_last, last, g_last, j_last)
    _wait_out(1 - s_last, last - 1, ((last - 1) // _GS) % 2, (last - 1) % _GS)
    _wait_out(s_last, last, g_last, j_last)


@functools.partial(
    pl.kernel,
    mesh=plsc.VectorSubcoreMesh(core_axis_name="c", subcore_axis_name="s"),
    out_type=[
        jax.ShapeDtypeStruct((_E, _D), jnp.float32),
        jax.ShapeDtypeStruct((_NC, _NA, _D), jnp.float32),
    ],
    scratch_types=_SC_SCRATCH,
)
def _sc_first(vnode_hbm, dst_hbm, vedge_hbm, oedge_hbm, part_hbm,
              idx, gbuf, eb0, eb1, acc, sem_g, sem_e, sem_o, sem_s, sem_i):
    cid = lax.axis_index("c")
    sid = lax.axis_index("s")
    wid = cid * _NS + sid
    row0 = sid * _RPS

    for sl in range(2):
        def _zrow(i, carry, _sl=sl):
            for j in range(_D // _L):
                gbuf[_sl, i, pl.ds(j * _L, _L)] = jnp.zeros((_L,), jnp.float32)
            return carry

        lax.fori_loop(0, _C, _zrow, 0)
    for r in range(_RPS // _C):
        pltpu.sync_copy(gbuf.at[r % 2], acc.at[pl.ds(row0 + r * _C, _C)])
    plsc.subcore_barrier()

    _sc_pipeline(_EPA // _C, _EPA // (_GS * _C), _EPA, 0,
                 vnode_hbm, dst_hbm, vedge_hbm, oedge_hbm,
                 idx, gbuf, (eb0, eb1), acc, sem_g, sem_e, sem_o, sem_s,
                 sem_i, wid)

    plsc.subcore_barrier()
    pltpu.sync_copy(acc.at[pl.ds(row0, _RPS)],
                    part_hbm.at[cid, pl.ds(row0, _RPS)])


@functools.partial(
    pl.kernel,
    mesh=plsc.VectorSubcoreMesh(core_axis_name="c", subcore_axis_name="s"),
    out_type=jax.ShapeDtypeStruct((_NC, _NA, _D), jnp.float32),
    scratch_types=_SC_SCRATCH,
)
def _sc_second(vnode_hbm, dst_hbm, vedge_hbm, part_hbm, oedge_hbm,
               partb_hbm,
               idx, gbuf, eb0, eb1, acc, sem_g, sem_e, sem_o, sem_s, sem_i):
    cid = lax.axis_index("c")
    sid = lax.axis_index("s")
    wid = cid * _NS + sid
    row0 = sid * _RPS

    pltpu.sync_copy(part_hbm.at[cid, pl.ds(row0, _RPS)],
                    acc.at[pl.ds(row0, _RPS)])
    plsc.subcore_barrier()

    _sc_pipeline(_EPB // _C, (_EPB // _C + _GS - 1) // _GS, _EPB, _EA,
                 vnode_hbm, dst_hbm, vedge_hbm, oedge_hbm,
                 idx, gbuf, (eb0, eb1), acc, sem_g, sem_e, sem_o, sem_s,
                 sem_i, wid)

    plsc.subcore_barrier()
    pltpu.sync_copy(acc.at[pl.ds(row0, _RPS)],
                    partb_hbm.at[cid, pl.ds(row0, _RPS)])


def _add_body(p_ref, o_ref):
    o_ref[...] = p_ref[0, :_N, :] + p_ref[1, :_N, :]


def _final_add(partials):
    return pl.pallas_call(
        _add_body,
        out_shape=jax.ShapeDtypeStruct((_N, _D), jnp.float32),
    )(partials)


def kernel(nodes, edges_index, edges_values,
           WQ_node_w, WQ_node_b, WQ_edge_w, WQ_edge_b,
           WK_node_w, WK_node_b, WK_edge_w, WK_edge_b,
           WV_node_w, WV_node_b, WV_edge_w, WV_edge_b):
    dst = edges_index[1].astype(jnp.int32)
    ga = _EPA // (_GS * _C)
    gb = (_EPB // _C + _GS - 1) // _GS
    dst_a = dst[:_EA].reshape(_NW, ga, _GS, _C)
    dst_b = jnp.pad(dst[_EA:].reshape(_NW, _EPB),
                    ((0, 0), (0, gb * _GS * _C - _EPB)))
    dst_b = dst_b.reshape(_NW, gb, _GS, _C)
    bn = WV_node_b.reshape(1, _D)
    be = WV_edge_b.reshape(1, _D)
    v_node, ve_a = _proj_a(nodes, edges_values, WV_node_w, bn, WV_edge_w, be)
    ve_b = _proj_b(edges_values, WV_edge_w, be)
    oedge_half, part_a = _sc_first(v_node, dst_a, ve_a)
    oedge_ref = jax.new_ref(oedge_half)
    part_b = _sc_second(v_node, dst_b, ve_b, part_a, oedge_ref)
    output_edges = jax.freeze(oedge_ref)
    output_nodes = _final_add(part_b)
    return (output_nodes, output_edges)


# speedup vs baseline: 1.0517x; 1.0011x over previous
import functools

import jax
import jax.numpy as jnp
from jax import lax
from jax.experimental import pallas as pl
from jax.experimental.pallas import tpu as pltpu
from jax.experimental.pallas import tpu_sc as plsc

_N = 10000
_E = 320000
_D = 128
_L = 16
_NC = 2
_NS = 16
_NW = _NC * _NS
_RB = 2560
_BA = 64
_EA = _BA * _RB
_EB = _E - _EA
_BB = _EB // _RB
_C = 80
_EPA = _EA // _NW
_EPB = _EB // _NW
_GS = 8
_NA = 10240
_RPS = _NA // _NS

_DN = (((1,), (1,)), ((), ()))


def _proj_a_body(nodes_ref, ev_ref, wn_ref, bn_ref, we_ref, be_ref,
                 vnode_ref, vedge_ref):
    @pl.when(pl.program_id(0) == 0)
    def _():
        vnode_ref[...] = (
            lax.dot_general(nodes_ref[...], wn_ref[...], _DN,
                            preferred_element_type=jnp.float32) + bn_ref[...]
        )
    vedge_ref[...] = (
        lax.dot_general(ev_ref[...], we_ref[...], _DN,
                        preferred_element_type=jnp.float32) + be_ref[...]
    )


def _proj_a(nodes, edges_values, wn, bn, we, be):
    return pl.pallas_call(
        _proj_a_body,
        grid=(_BA,),
        in_specs=[
            pl.BlockSpec((_N, _D), lambda i: (0, 0)),
            pl.BlockSpec((_RB, _D), lambda i: (i, 0)),
            pl.BlockSpec((_D, _D), lambda i: (0, 0)),
            pl.BlockSpec((1, _D), lambda i: (0, 0)),
            pl.BlockSpec((_D, _D), lambda i: (0, 0)),
            pl.BlockSpec((1, _D), lambda i: (0, 0)),
        ],
        out_specs=[
            pl.BlockSpec((_N, _D), lambda i: (0, 0)),
            pl.BlockSpec((_RB, _D), lambda i: (i, 0)),
        ],
        out_shape=[
            jax.ShapeDtypeStruct((_N, _D), jnp.float32),
            jax.ShapeDtypeStruct((_EA, _D), jnp.float32),
        ],
    )(nodes, edges_values, wn, bn, we, be)


def _proj_b_body(ev_ref, we_ref, be_ref, vedge_ref):
    vedge_ref[...] = (
        lax.dot_general(ev_ref[...], we_ref[...], _DN,
                        preferred_element_type=jnp.float32) + be_ref[...]
    )


def _proj_b(edges_values, we, be):
    return pl.pallas_call(
        _proj_b_body,
        grid=(_BB,),
        in_specs=[
            pl.BlockSpec((_RB, _D), lambda i: (i + _BA, 0)),
            pl.BlockSpec((_D, _D), lambda i: (0, 0)),
            pl.BlockSpec((1, _D), lambda i: (0, 0)),
        ],
        out_specs=pl.BlockSpec((_RB, _D), lambda i: (i, 0)),
        out_shape=jax.ShapeDtypeStruct((_EB, _D), jnp.float32),
    )(edges_values, we, be)


_SC_SCRATCH = [
    pltpu.VMEM((2, _GS, _C), jnp.int32),
    pltpu.VMEM((2, _C, _D), jnp.float32),
    pltpu.VMEM((_C, _D), jnp.float32),
    pltpu.VMEM((_C, _D), jnp.float32),
    pltpu.VMEM_SHARED((_NA, _D), jnp.float32),
    pltpu.SemaphoreType.DMA((2,)),
    pltpu.SemaphoreType.DMA((2,)),
    pltpu.SemaphoreType.DMA((2,)),
    pltpu.SemaphoreType.DMA((2,)),
    pltpu.SemaphoreType.DMA((2,)),
]


def _sc_pipeline(nchunk, ngroups, epw, half_base,
                 vnode_hbm, dst_hbm, vedge_hbm, oedge_hbm,
                 idx, gbuf, ebufs, acc, sem_g, sem_e, sem_o, sem_s,
                 sem_i, wid):
    ebase = wid * epw

    def _start_idx(s, g):
        pltpu.async_copy(dst_hbm.at[wid, g], idx.at[s], sem_i.at[s])

    def _wait_idx(s, g):
        pltpu.make_async_copy(dst_hbm.at[wid, g], idx.at[s],
                              sem_i.at[s]).wait()

    def _start_in(s, i, gs, j):
        pltpu.async_copy(vnode_hbm.at[idx.at[gs, j]], gbuf.at[s],
                         sem_g.at[s])
        pltpu.async_copy(vedge_hbm.at[pl.ds(ebase + i * _C, _C)],
                         ebufs[s], sem_e.at[s])

    def _wait_in(s, i, gs, j):
        pltpu.make_async_copy(vnode_hbm.at[idx.at[gs, j]], gbuf.at[s],
                              sem_g.at[s]).wait()
        pltpu.make_async_copy(vedge_hbm.at[pl.ds(ebase + i * _C, _C)],
                              ebufs[s], sem_e.at[s]).wait()

    def _mul(s):
        ebuf = ebufs[s]

        @plsc.parallel_loop(0, _C, step=1, unroll=4)
        def _mrow(r2):
            for j in range(_D // _L):
                sl = pl.ds(j * _L, _L)
                gbuf[s, r2, sl] = gbuf[s, r2, sl] * ebuf[r2, sl]

    def _start_out(s, i, gs, j):
        pltpu.async_copy(gbuf.at[s],
                         oedge_hbm.at[pl.ds(half_base + ebase + i * _C, _C)],
                         sem_o.at[s])
        pltpu.async_copy(gbuf.at[s], acc.at[idx.at[gs, j]], sem_s.at[s],
                         add=True)

    def _wait_out(s, i, gs, j):
        pltpu.make_async_copy(gbuf.at[s],
                              oedge_hbm.at[pl.ds(half_base + ebase + i * _C,
                                                 _C)],
                              sem_o.at[s]).wait()
        pltpu.make_async_copy(gbuf.at[s], acc.at[idx.at[gs, j]],
                              sem_s.at[s]).wait()

    pltpu.sync_copy(dst_hbm.at[wid, 0], idx.at[0])
    _start_in(0, 0, 0, 0)

    def _chunk_step(i, s):
        ns = 1 - s
        g = lax.div(i, _GS)
        j = lax.rem(i, _GS)
        gs = lax.rem(g, 2)
        ngs = 1 - gs

        @pl.when(i >= 1)
        def _():
            _wait_out(ns, i - 1, lax.rem(lax.div(i - 1, _GS), 2),
                      lax.rem(i - 1, _GS))

        @pl.when(jnp.logical_and(j == 1, g < ngroups - 1))
        def _():
            _start_idx(ngs, g + 1)

        @pl.when(j == _GS - 1)
        def _():
            _wait_idx(ngs, g + 1)

        nxt_gs = lax.rem(lax.div(i + 1, _GS), 2)
        _start_in(ns, i + 1, nxt_gs, lax.rem(i + 1, _GS))
        _wait_in(s, i, gs, j)
        _mul(s)
        _start_out(s, i, gs, j)

    nbody = nchunk - 1

    def _pair(p, carry):
        _chunk_step(2 * p, 0)
        _chunk_step(2 * p + 1, 1)
        return carry

    lax.fori_loop(0, nbody // 2, _pair, 0)
    if nbody % 2:
        _chunk_step(nbody - 1, (nbody - 1) % 2)
    last = nchunk - 1
    s_last = last % 2
    g_last = (last // _GS) % 2
    j_last = last % _GS
    _wait_in(s_last, last, g_last, j_last)
    _mul(s_last)
    _start_out(s_last, last, g_last, j_last)
    _wait_out(1 - s_last, last - 1, ((last - 1) // _GS) % 2, (last - 1) % _GS)
    _wait_out(s_last, last, g_last, j_last)


@functools.partial(
    pl.kernel,
    mesh=plsc.VectorSubcoreMesh(core_axis_name="c", subcore_axis_name="s"),
    out_type=[
        jax.ShapeDtypeStruct((_E, _D), jnp.float32),
        jax.ShapeDtypeStruct((_NC, _NA, _D), jnp.float32),
    ],
    scratch_types=_SC_SCRATCH,
)
def _sc_first(vnode_hbm, dst_hbm, vedge_hbm, oedge_hbm, part_hbm,
              idx, gbuf, eb0, eb1, acc, sem_g, sem_e, sem_o, sem_s, sem_i):
    cid = lax.axis_index("c")
    sid = lax.axis_index("s")
    wid = cid * _NS + sid
    row0 = sid * _RPS

    for sl in range(2):
        def _zrow(i, carry, _sl=sl):
            for j in range(_D // _L):
                gbuf[_sl, i, pl.ds(j * _L, _L)] = jnp.zeros((_L,), jnp.float32)
            return carry

        lax.fori_loop(0, _C, _zrow, 0)
    for r in range(_RPS // _C):
        pltpu.sync_copy(gbuf.at[r % 2], acc.at[pl.ds(row0 + r * _C, _C)])
    plsc.subcore_barrier()

    _sc_pipeline(_EPA // _C, _EPA // (_GS * _C), _EPA, 0,
                 vnode_hbm, dst_hbm, vedge_hbm, oedge_hbm,
                 idx, gbuf, (eb0, eb1), acc, sem_g, sem_e, sem_o, sem_s,
                 sem_i, wid)

    plsc.subcore_barrier()
    pltpu.sync_copy(acc.at[pl.ds(row0, _RPS)],
                    part_hbm.at[cid, pl.ds(row0, _RPS)])


@functools.partial(
    pl.kernel,
    mesh=plsc.VectorSubcoreMesh(core_axis_name="c", subcore_axis_name="s"),
    out_type=jax.ShapeDtypeStruct((_NC, _NA, _D), jnp.float32),
    scratch_types=_SC_SCRATCH,
)
def _sc_second(vnode_hbm, dst_hbm, vedge_hbm, part_hbm, oedge_hbm,
               partb_hbm,
               idx, gbuf, eb0, eb1, acc, sem_g, sem_e, sem_o, sem_s, sem_i):
    cid = lax.axis_index("c")
    sid = lax.axis_index("s")
    wid = cid * _NS + sid
    row0 = sid * _RPS

    pltpu.sync_copy(part_hbm.at[cid, pl.ds(row0, _RPS)],
                    acc.at[pl.ds(row0, _RPS)])
    plsc.subcore_barrier()

    _sc_pipeline(_EPB // _C, (_EPB // _C + _GS - 1) // _GS, _EPB, _EA,
                 vnode_hbm, dst_hbm, vedge_hbm, oedge_hbm,
                 idx, gbuf, (eb0, eb1), acc, sem_g, sem_e, sem_o, sem_s,
                 sem_i, wid)

    plsc.subcore_barrier()
    pltpu.sync_copy(acc.at[pl.ds(row0, _RPS)],
                    partb_hbm.at[cid, pl.ds(row0, _RPS)])


def _add_body(p_ref, o_ref):
    o_ref[...] = p_ref[0, :_N, :] + p_ref[1, :_N, :]


def _final_add(partials):
    return pl.pallas_call(
        _add_body,
        out_shape=jax.ShapeDtypeStruct((_N, _D), jnp.float32),
    )(partials)


def kernel(nodes, edges_index, edges_values,
           WQ_node_w, WQ_node_b, WQ_edge_w, WQ_edge_b,
           WK_node_w, WK_node_b, WK_edge_w, WK_edge_b,
           WV_node_w, WV_node_b, WV_edge_w, WV_edge_b):
    dst = edges_index[1].astype(jnp.int32)
    ga = _EPA // (_GS * _C)
    gb = (_EPB // _C + _GS - 1) // _GS
    dst_a = dst[:_EA].reshape(_NW, ga, _GS, _C)
    dst_b = jnp.pad(dst[_EA:].reshape(_NW, _EPB),
                    ((0, 0), (0, gb * _GS * _C - _EPB)))
    dst_b = dst_b.reshape(_NW, gb, _GS, _C)
    bn = WV_node_b.reshape(1, _D)
    be = WV_edge_b.reshape(1, _D)
    v_node, ve_a = _proj_a(nodes, edges_values, WV_node_w, bn, WV_edge_w, be)
    ve_b = _proj_b(edges_values, WV_edge_w, be)
    oedge_half, part_a = _sc_first(v_node, dst_a, ve_a)
    oedge_ref = jax.new_ref(oedge_half)
    part_b = _sc_second(v_node, dst_b, ve_b, part_a, oedge_ref)
    output_edges = jax.freeze(oedge_ref)
    output_nodes = _final_add(part_b)
    return (output_nodes, output_edges)
